# baseline (speedup 1.0000x reference)
import functools

import jax
import jax.numpy as jnp
from jax import lax
from jax.experimental import pallas as pl
from jax.experimental.pallas import tpu as pltpu
from jax.experimental.pallas import tpu_sc as plsc

N = 10000
E = 320000
D_IN = 128
D_HID = 128
D_OUT = 16

NC = 2
NS = 16
NW = NC * NS
L = 16

EPT = 10240
E_PAD = NW * EPT

N_PAD = 10240
ZR = 8


@functools.cache
def _mesh():
    return plsc.VectorSubcoreMesh(
        core_axis_name="c", subcore_axis_name="s",
        num_cores=NC, num_subcores=NS)


_GATHER_DN = lax.GatherDimensionNumbers(
    offset_dims=(), collapsed_slice_dims=(0,), start_index_map=(0,))


def _splat(vec, lane):
    idx = jnp.full((L, 1), lane, jnp.int32)
    return lax.gather(vec, idx, _GATHER_DN, (1,),
                      mode=lax.GatherScatterMode.PROMISE_IN_BOUNDS)


def _zero_ref(ref, n_elems):
    z = jnp.zeros((L,), jnp.float32)
    for i in range(n_elems // L):
        ref[pl.ds(i * L, L)] = z


K_DEG, C_DEG = 64, 160


@functools.cache
def _make_deg_kernel():
    return functools.partial(
        pl.kernel,
        out_type=jax.ShapeDtypeStruct((NC, N_PAD), jnp.float32),
        mesh=_mesh(),
        scratch_types=[
            pltpu.VMEM((C_DEG, K_DEG), jnp.int32),
            pltpu.VMEM((C_DEG, K_DEG), jnp.float32),
            pltpu.VMEM((N_PAD // NS,), jnp.float32),
            pltpu.VMEM_SHARED((N_PAD,), jnp.float32),
        ],
    )(_deg_body)


def _deg_body(dst_hbm, ew_hbm, out_hbm, dst_v, ew_v, zb, deg_sh):
    cid = lax.axis_index("c")
    sid = lax.axis_index("s")
    wid = sid * NC + cid
    seg = N_PAD // NS
    _zero_ref(zb, seg)
    base = pl.multiple_of(sid * seg, 8)
    pltpu.sync_copy(zb, deg_sh.at[pl.ds(base, seg)])
    pltpu.sync_copy(dst_hbm.at[wid], dst_v)
    pltpu.sync_copy(ew_hbm.at[wid], ew_v)
    plsc.subcore_barrier()

    def body(j, _):
        pltpu.sync_copy(ew_v.at[j], deg_sh.at[dst_v.at[j]], add=True)
        return 0

    lax.fori_loop(0, C_DEG, body, 0)
    plsc.subcore_barrier()
    pltpu.sync_copy(deg_sh.at[pl.ds(base, seg)],
                    out_hbm.at[cid, pl.ds(base, seg)])


@functools.cache
def _make_agg_kernel(D, K, C, G, packed=False):
    in_cols = D // 2 if packed else D
    in_dtype = jnp.int32 if packed else jnp.float32

    @functools.partial(
        pl.kernel,
        out_type=jax.ShapeDtypeStruct((NC, N_PAD, D), jnp.float32),
        mesh=_mesh(),
        compiler_params=pltpu.CompilerParams(
            needs_layout_passes=False,
            use_tc_tiling_on_sc=(in_cols % 128 == 0)),
        scratch_types=[
            pltpu.VMEM((G, K), jnp.int32),
            pltpu.VMEM((G, K), jnp.int32),
            pltpu.VMEM((G, K), jnp.float32),
            pltpu.VMEM((2, K, in_cols), in_dtype),
            pltpu.VMEM((2, K, D) if packed else (16,), jnp.float32),
            pltpu.VMEM_SHARED((N_PAD, D), jnp.float32),
            pltpu.SemaphoreType.DMA,
            pltpu.SemaphoreType.DMA,
            pltpu.SemaphoreType.DMA,
            pltpu.SemaphoreType.DMA,
        ],
    )
    def agg(h_hbm, src_hbm, dst_hbm, ew_hbm, z_hbm, out_hbm,
            src_v, dst_v, ew_v, rows_v, rows_out, acc_sh,
            sem0, sem1, ssem0, ssem1):
        cid = lax.axis_index("c")
        sid = lax.axis_index("s")
        wid = sid * NC + cid
        rows_per_tile = N_PAD // NS
        row0 = sid * rows_per_tile
        sems = (sem0, sem1)
        ssems = (ssem0, ssem1)

        pltpu.sync_copy(z_hbm.at[pl.ds(row0, rows_per_tile)],
                        acc_sh.at[pl.ds(row0, rows_per_tile)])
        plsc.subcore_barrier()

        def gather_desc(jj, s):
            return pltpu.make_async_copy(
                h_hbm.at[src_v.at[jj]], rows_v.at[s], sems[s])

        out_rows = rows_out if packed else rows_v

        def scatter_desc(jj, s):
            return pltpu.make_async_copy(
                out_rows.at[s], acc_sh.at[dst_v.at[jj]], ssems[s])

        def scatter_start(jj, s):
            pltpu.async_copy(out_rows.at[s], acc_sh.at[dst_v.at[jj]],
                             ssems[s], add=True)

        def scale(jj, s):
            erow = ew_v.at[jj]
            for g in range(K // L):
                c16 = erow[pl.ds(g * L, L)]
                for kk in range(L):
                    k = g * L + kk
                    spl = _splat(c16, kk)
                    if packed:
                        for q in range(D // (2 * L)):
                            x = rows_v[s, k, pl.ds(q * L, L)]
                            ev = plsc.bitcast(
                                lax.shift_left(x, jnp.int32(16)), jnp.float32)
                            od = plsc.bitcast(
                                lax.bitwise_and(x, jnp.int32(-65536)),
                                jnp.float32)
                            rows_out[s, k, pl.ds(2 * q * L, L)] = ev * spl
                            rows_out[s, k, pl.ds((2 * q + 1) * L, L)] = od * spl
                    else:
                        for l in range(D // L):
                            sl = pl.ds(l * L, L)
                            rows_v[s, k, sl] = rows_v[s, k, sl] * spl

        def pair_body(m, _):
            j0 = m * 2
            gather_desc(j0, 0).wait()

            @pl.when(m > 0)
            def _():
                scatter_desc(j0 - 1, 1).wait()

            gather_desc(j0 + 1, 1).start()
            scale(j0, 0)
            scatter_start(j0, 0)
            gather_desc(j0 + 1, 1).wait()
            scatter_desc(j0, 0).wait()

            @pl.when(m < G // 2 - 1)
            def _():
                gather_desc(j0 + 2, 0).start()

            scale(j0 + 1, 1)
            scatter_start(j0 + 1, 1)
            return 0

        def group_body(gi, _):
            g0 = pl.multiple_of(gi * G, 8)
            pltpu.sync_copy(src_hbm.at[wid, pl.ds(g0, G)], src_v)
            pltpu.sync_copy(dst_hbm.at[wid, pl.ds(g0, G)], dst_v)
            pltpu.sync_copy(ew_hbm.at[wid, pl.ds(g0, G)], ew_v)
            gather_desc(0, 0).start()
            lax.fori_loop(0, G // 2, pair_body, 0)
            scatter_desc(G - 1, 1).wait()
            return 0

        lax.fori_loop(0, C // G, group_body, 0)
        plsc.subcore_barrier()
        pltpu.sync_copy(acc_sh.at[pl.ds(row0, rows_per_tile)],
                        out_hbm.at[cid, pl.ds(row0, rows_per_tile)])

    return agg


def _tc_a_body(degp_ref, x_ref, w1_ref, dis_ref, hw1_ref):
    deg = degp_ref[0] + degp_ref[1] + 1.0
    dis = jnp.where(deg > 0, lax.rsqrt(deg), 0.0)
    dis_ref[...] = dis
    hw1 = lax.dot_general(
        x_ref[...], w1_ref[...], (((1,), (1,)), ((), ())),
        precision=lax.Precision.HIGHEST,
        preferred_element_type=jnp.float32)
    hw1_ref[...] = dis[:N] * hw1


def _tc_b_body(a_ref, hw1_ref, dis_ref, b1_ref, w2_ref, hw2_ref):
    dis = dis_ref[...]
    a = a_ref[...]
    h = a[0, :N] + a[1, :N] + hw1_ref[...]
    h = jnp.maximum(dis * h + b1_ref[...], 0.0)
    hw2 = lax.dot_general(
        h, w2_ref[...], (((1,), (1,)), ((), ())),
        precision=lax.Precision.HIGHEST,
        preferred_element_type=jnp.float32)
    hw2_ref[...] = dis * hw2


def _tc_c_body(a_ref, hw2_ref, dis_ref, b2_ref, out_ref):
    dis = dis_ref[...]
    a = a_ref[...]
    out_ref[...] = (dis * (a[0, :N] + a[1, :N] + hw2_ref[...])
                    + b2_ref[...])


_PERM = []
for _g in range(D_IN // 32):
    for _i in range(16):
        _PERM.extend((32 * _g + _i, 32 * _g + 16 + _i))
_PERM = tuple(_PERM)


def kernel(x, train_pos_edge_index, edge_weight, W1, b1, W2, b2):
    src = train_pos_edge_index[0].astype(jnp.int32)
    dst = train_pos_edge_index[1].astype(jnp.int32)
    ew = edge_weight[:, 0].astype(jnp.float32)
    pad = E_PAD - E
    src_f = jnp.pad(src, (0, pad))
    dst_f = jnp.pad(dst, (0, pad))
    ew_f = jnp.pad(ew, (0, pad))

    def views(K, C):
        return (src_f.reshape(NW, C, K), dst_f.reshape(NW, C, K),
                ew_f.reshape(NW, C, K))

    deg_parts = _make_deg_kernel()(*views(K_DEG, C_DEG)[1:3])

    dis_pad, hw1 = pl.pallas_call(
        _tc_a_body,
        out_shape=[
            jax.ShapeDtypeStruct((N_PAD, 1), jnp.float32),
            jax.ShapeDtypeStruct((N, D_IN), jnp.float32),
        ],
    )(deg_parts[:, :, None], x, W1)
    dis_col = dis_pad[:N]

    hw1_bits = lax.bitcast_convert_type(
        hw1[:, jnp.asarray(_PERM)].astype(jnp.bfloat16).reshape(
            N, D_HID // 2, 2),
        jnp.int32)

    s1, d1, e1 = views(80, 128)
    acc1 = _make_agg_kernel(D_HID, 80, 128, 16, True)(
        hw1_bits, s1, d1, e1, jnp.zeros((N_PAD, D_HID), jnp.float32))

    hw2 = pl.pallas_call(
        _tc_b_body,
        out_shape=jax.ShapeDtypeStruct((N, D_OUT), jnp.float32),
    )(acc1, hw1, dis_col, b1.reshape(1, D_HID), W2)

    s2, d2, e2 = views(128, 80)
    acc2 = _make_agg_kernel(D_OUT, 128, 80, 8, False)(
        hw2, s2, d2, e2, jnp.zeros((N_PAD, D_OUT), jnp.float32))

    out = dis_col * (acc2[0, :N] + acc2[1, :N] + hw2) + b2
    return out


# speedup vs baseline: 1.0994x; 1.0994x over previous
import functools

import jax
import jax.numpy as jnp
from jax import lax
from jax.experimental import pallas as pl
from jax.experimental.pallas import tpu as pltpu
from jax.experimental.pallas import tpu_sc as plsc

N = 10000
E = 320000
D_IN = 128
D_HID = 128
D_OUT = 16

NC = 2
NS = 16
NW = NC * NS
L = 16

EPT = 10240
E_PAD = NW * EPT

N_PAD = 10240


@functools.cache
def _mesh():
    return plsc.VectorSubcoreMesh(
        core_axis_name="c", subcore_axis_name="s",
        num_cores=NC, num_subcores=NS)


_GATHER_DN = lax.GatherDimensionNumbers(
    offset_dims=(), collapsed_slice_dims=(0,), start_index_map=(0,))


def _splat(vec, lane):
    idx = jnp.full((L, 1), lane, jnp.int32)
    return lax.gather(vec, idx, _GATHER_DN, (1,),
                      mode=lax.GatherScatterMode.PROMISE_IN_BOUNDS)


def _zero_ref(ref, n_elems):
    z = jnp.zeros((L,), jnp.float32)
    for i in range(n_elems // L):
        ref[pl.ds(i * L, L)] = z


K_DEG, C_DEG = 64, 160


@functools.cache
def _make_deg_kernel():
    return functools.partial(
        pl.kernel,
        out_type=jax.ShapeDtypeStruct((NC, N_PAD), jnp.float32),
        mesh=_mesh(),
        scratch_types=[
            pltpu.VMEM((C_DEG, K_DEG), jnp.int32),
            pltpu.VMEM((C_DEG, K_DEG), jnp.float32),
            pltpu.VMEM((N_PAD // NS,), jnp.float32),
            pltpu.VMEM_SHARED((N_PAD,), jnp.float32),
        ],
    )(_deg_body)


def _deg_body(dst_hbm, ew_hbm, out_hbm, dst_v, ew_v, zb, deg_sh):
    cid = lax.axis_index("c")
    sid = lax.axis_index("s")
    wid = sid * NC + cid
    seg = N_PAD // NS
    _zero_ref(zb, seg)
    base = pl.multiple_of(sid * seg, 8)
    pltpu.sync_copy(zb, deg_sh.at[pl.ds(base, seg)])
    pltpu.sync_copy(dst_hbm.at[wid], dst_v)
    pltpu.sync_copy(ew_hbm.at[wid], ew_v)
    plsc.subcore_barrier()

    def body(j, _):
        pltpu.sync_copy(ew_v.at[j], deg_sh.at[dst_v.at[j]], add=True)
        return 0

    lax.fori_loop(0, C_DEG, body, 0)
    plsc.subcore_barrier()
    pltpu.sync_copy(deg_sh.at[pl.ds(base, seg)],
                    out_hbm.at[cid, pl.ds(base, seg)])


@functools.cache
def _make_agg_kernel(D, K, C, G, packed=False):
    in_cols = D // 2 if packed else D
    in_dtype = jnp.int32 if packed else jnp.float32

    @functools.partial(
        pl.kernel,
        out_type=jax.ShapeDtypeStruct((NC, N_PAD, D), jnp.float32),
        mesh=_mesh(),
        compiler_params=pltpu.CompilerParams(
            needs_layout_passes=False,
            use_tc_tiling_on_sc=(in_cols % 128 == 0)),
        scratch_types=[
            pltpu.VMEM((G, K), jnp.int32),
            pltpu.VMEM((G, K), jnp.int32),
            pltpu.VMEM((G, K), jnp.float32),
            pltpu.VMEM((2, K, in_cols), in_dtype),
            pltpu.VMEM((2, K, D) if packed else (16,), jnp.float32),
            pltpu.VMEM_SHARED((N_PAD, D), jnp.float32),
            pltpu.SemaphoreType.DMA,
            pltpu.SemaphoreType.DMA,
            pltpu.SemaphoreType.DMA,
            pltpu.SemaphoreType.DMA,
        ],
    )
    def agg(h_hbm, src_hbm, dst_hbm, ew_hbm, z_hbm, out_hbm,
            src_v, dst_v, ew_v, rows_v, rows_out, acc_sh,
            sem0, sem1, ssem0, ssem1):
        cid = lax.axis_index("c")
        sid = lax.axis_index("s")
        wid = sid * NC + cid
        rows_per_tile = N_PAD // NS
        row0 = sid * rows_per_tile
        sems = (sem0, sem1)
        ssems = (ssem0, ssem1)

        pltpu.sync_copy(z_hbm.at[pl.ds(row0, rows_per_tile)],
                        acc_sh.at[pl.ds(row0, rows_per_tile)])
        plsc.subcore_barrier()

        def gather_desc(jj, s):
            return pltpu.make_async_copy(
                h_hbm.at[src_v.at[jj]], rows_v.at[s], sems[s])

        out_rows = rows_out if packed else rows_v

        def scatter_desc(jj, s):
            return pltpu.make_async_copy(
                out_rows.at[s], acc_sh.at[dst_v.at[jj]], ssems[s])

        def scatter_start(jj, s):
            pltpu.async_copy(out_rows.at[s], acc_sh.at[dst_v.at[jj]],
                             ssems[s], add=True)

        def scale(jj, s):
            erow = ew_v.at[jj]
            for g in range(K // L):
                c16 = erow[pl.ds(g * L, L)]
                for kk in range(L):
                    k = g * L + kk
                    spl = _splat(c16, kk)
                    if packed:
                        for q in range(D // (2 * L)):
                            x = rows_v[s, k, pl.ds(q * L, L)]
                            ev = plsc.bitcast(
                                lax.shift_left(x, jnp.int32(16)), jnp.float32)
                            od = plsc.bitcast(
                                lax.bitwise_and(x, jnp.int32(-65536)),
                                jnp.float32)
                            rows_out[s, k, pl.ds(2 * q * L, L)] = ev * spl
                            rows_out[s, k, pl.ds((2 * q + 1) * L, L)] = od * spl
                    else:
                        for l in range(D // L):
                            sl = pl.ds(l * L, L)
                            rows_v[s, k, sl] = rows_v[s, k, sl] * spl

        def pair_body(m, _):
            j0 = m * 2
            gather_desc(j0, 0).wait()

            @pl.when(m > 0)
            def _():
                scatter_desc(j0 - 1, 1).wait()

            gather_desc(j0 + 1, 1).start()
            scale(j0, 0)
            scatter_start(j0, 0)
            gather_desc(j0 + 1, 1).wait()
            scatter_desc(j0, 0).wait()

            @pl.when(m < G // 2 - 1)
            def _():
                gather_desc(j0 + 2, 0).start()

            scale(j0 + 1, 1)
            scatter_start(j0 + 1, 1)
            return 0

        def group_body(gi, _):
            g0 = pl.multiple_of(gi * G, 8)
            pltpu.sync_copy(src_hbm.at[wid, pl.ds(g0, G)], src_v)
            pltpu.sync_copy(dst_hbm.at[wid, pl.ds(g0, G)], dst_v)
            pltpu.sync_copy(ew_hbm.at[wid, pl.ds(g0, G)], ew_v)
            gather_desc(0, 0).start()
            lax.fori_loop(0, G // 2, pair_body, 0)
            scatter_desc(G - 1, 1).wait()
            return 0

        lax.fori_loop(0, C // G, group_body, 0)
        plsc.subcore_barrier()
        pltpu.sync_copy(acc_sh.at[pl.ds(row0, rows_per_tile)],
                        out_hbm.at[cid, pl.ds(row0, rows_per_tile)])

    return agg


def _tc_a_body(degp_ref, x_ref, w1_ref, dis_ref, hw1_ref):
    deg = degp_ref[0] + degp_ref[1] + 1.0
    dis = jnp.where(deg > 0, lax.rsqrt(deg), 0.0)
    dis_ref[...] = dis
    hw1 = lax.dot_general(
        x_ref[...], w1_ref[...], (((1,), (1,)), ((), ())),
        precision=lax.Precision.HIGHEST,
        preferred_element_type=jnp.float32)
    hw1_ref[...] = dis[:N] * hw1


def _tc_b_body(a_ref, hw1_ref, dis_ref, b1_ref, w2_ref, hw2_ref):
    dis = dis_ref[...]
    a = a_ref[...]
    h = a[0, :N] + a[1, :N] + hw1_ref[...]
    h = jnp.maximum(dis * h + b1_ref[...], 0.0)
    hw2 = lax.dot_general(
        h, w2_ref[...], (((1,), (1,)), ((), ())),
        precision=lax.Precision.HIGHEST,
        preferred_element_type=jnp.float32)
    hw2_ref[...] = dis * hw2


def _tc_c_body(a_ref, hw2_ref, dis_ref, b2_ref, out_ref):
    dis = dis_ref[...]
    a = a_ref[...]
    out_ref[...] = (dis * (a[0, :N] + a[1, :N] + hw2_ref[...])
                    + b2_ref[...])


_PERM = []
for _g in range(D_IN // 32):
    for _i in range(16):
        _PERM.extend((32 * _g + _i, 32 * _g + 16 + _i))
_PERM = tuple(_PERM)


def kernel(x, train_pos_edge_index, edge_weight, W1, b1, W2, b2):
    src = train_pos_edge_index[0].astype(jnp.int32)
    dst = train_pos_edge_index[1].astype(jnp.int32)
    ew = edge_weight[:, 0].astype(jnp.float32)
    pad = E_PAD - E
    src_f = jnp.pad(src, (0, pad))
    dst_f = jnp.pad(dst, (0, pad))
    ew_f = jnp.pad(ew, (0, pad))

    def views(K, C):
        return (src_f.reshape(NW, C, K), dst_f.reshape(NW, C, K),
                ew_f.reshape(NW, C, K))

    deg_parts = _make_deg_kernel()(*views(K_DEG, C_DEG)[1:3])

    dis_pad, hw1 = pl.pallas_call(
        _tc_a_body,
        out_shape=[
            jax.ShapeDtypeStruct((N_PAD, 1), jnp.float32),
            jax.ShapeDtypeStruct((N, D_IN), jnp.float32),
        ],
    )(deg_parts[:, :, None], x, W1)
    dis_col = dis_pad[:N]

    hw1_bits = lax.bitcast_convert_type(
        hw1[:, jnp.asarray(_PERM)].astype(jnp.bfloat16).reshape(
            N, D_HID // 2, 2),
        jnp.int32)

    s1, d1, e1 = views(80, 128)
    acc1 = _make_agg_kernel(D_HID, 80, 128, 16, True)(
        hw1_bits, s1, d1, e1, jnp.zeros((N_PAD, D_HID), jnp.float32))

    hw2 = pl.pallas_call(
        _tc_b_body,
        out_shape=jax.ShapeDtypeStruct((N, D_OUT), jnp.float32),
    )(acc1, hw1, dis_col, b1.reshape(1, D_HID), W2)

    s2, d2, e2 = views(128, 80)
    acc2 = _make_agg_kernel(D_OUT, 128, 80, 8, False)(
        hw2, s2, d2, e2, jnp.zeros((N_PAD, D_OUT), jnp.float32))

    out = pl.pallas_call(
        _tc_c_body,
        out_shape=jax.ShapeDtypeStruct((N, D_OUT), jnp.float32),
    )(acc2, hw2, dis_col, b2.reshape(1, D_OUT))
    return out
